# Initial kernel scaffold; baseline (speedup 1.0000x reference)
import jax
import jax.numpy as jnp
from jax.experimental import pallas as pl


def kernel(x, edge_index, batch, We1, be1, We2, be2, Wg, bg, gamma, beta, Wr0, br0, Wr1, br1, Wr2, br2):
    raise NotImplementedError("write your pallas kernel here")


# speedup vs baseline: 8.2915x; 8.2915x over previous
import functools

import jax
import jax.numpy as jnp
from jax import lax
from jax.experimental import pallas as pl
from jax.experimental.pallas import tpu as pltpu
from jax.experimental.pallas import tpu_sc as plsc

N = 10000
D = 128
E = 320000
NG = 64
NL = 3
NC = 2
NS = 16
NW = NC * NS
EPW = E // NW
CH = 80
NCH = EPW // CH
RPT = N // NS


@functools.cache
def _sc_kernels():
  mesh = plsc.VectorSubcoreMesh(core_axis_name="c", subcore_axis_name="s",
                                num_cores=NC, num_subcores=NS)

  @functools.partial(
      pl.kernel,
      out_type=jax.ShapeDtypeStruct((NW, RPT, D), jnp.float32),
      mesh=mesh,
      scratch_types=[
          pltpu.VMEM((CH,), jnp.int32),
          pltpu.VMEM((CH, D), jnp.float32),
          pltpu.VMEM_SHARED((N, D), jnp.float32),
      ],
  )
  def _deg_sc(dst_hbm, zeros_hbm, ones_hbm, out_hbm, didx, ones_v, deg_sh):
      c = lax.axis_index("c")
      s = lax.axis_index("s")
      wid = c * NS + s
      pltpu.sync_copy(zeros_hbm.at[s], deg_sh.at[pl.ds(s * RPT, RPT)])
      pltpu.sync_copy(ones_hbm, ones_v)
      plsc.subcore_barrier()
      base = wid * EPW

      def body(i, carry):
        pltpu.sync_copy(dst_hbm.at[pl.ds(base + i * CH, CH)], didx)
        pltpu.sync_copy(ones_v, deg_sh.at[didx], add=True)
        return carry

      lax.fori_loop(0, NCH, body, 0)
      plsc.subcore_barrier()
      pltpu.sync_copy(deg_sh.at[pl.ds(s * RPT, RPT)], out_hbm.at[wid])

  @functools.partial(
      pl.kernel,
      out_type=jax.ShapeDtypeStruct((NW, RPT, D), jnp.float32),
      mesh=mesh,
      scratch_types=[
          pltpu.VMEM((CH,), jnp.int32),
          pltpu.VMEM((CH,), jnp.int32),
          pltpu.VMEM((CH, D), jnp.float32),
          pltpu.VMEM_SHARED((N, D), jnp.float32),
          pltpu.SemaphoreType.DMA,
      ],
  )
  def _agg_sc(m_hbm, src_hbm, dst_hbm, zeros_hbm, out_hbm,
              sidx, didx, rows, agg_sh, gsem):
      c = lax.axis_index("c")
      s = lax.axis_index("s")
      wid = c * NS + s
      pltpu.sync_copy(zeros_hbm.at[s], agg_sh.at[pl.ds(s * RPT, RPT)])
      plsc.subcore_barrier()
      base = wid * EPW

      def body(i, carry):
        pltpu.sync_copy(src_hbm.at[pl.ds(base + i * CH, CH)], sidx)
        pltpu.sync_copy(dst_hbm.at[pl.ds(base + i * CH, CH)], didx)
        pltpu.async_copy(m_hbm.at[sidx], rows, gsem).wait()
        pltpu.sync_copy(rows, agg_sh.at[didx], add=True)
        return carry

      lax.fori_loop(0, NCH, body, 0)
      plsc.subcore_barrier()
      pltpu.sync_copy(agg_sh.at[pl.ds(s * RPT, RPT)], out_hbm.at[wid])

  return _deg_sc, _agg_sc


def _enc_body(x_ref, we1_ref, be1_ref, we2_ref, be2_ref, wg0_ref, degp_ref,
              h_ref, m0_ref, dinv_ref):
    x = x_ref[...]
    h1 = jnp.maximum(
        jnp.dot(x, we1_ref[...], preferred_element_type=jnp.float32)
        + be1_ref[...], 0.0)
    h = (jnp.dot(h1, we2_ref[...], preferred_element_type=jnp.float32)
         + be2_ref[...])
    degw = degp_ref[0] + degp_ref[1]
    deg = jnp.sum(degw, axis=1, keepdims=True) * (1.0 / D)
    dinv = jnp.where(deg > 0.0, lax.rsqrt(jnp.maximum(deg, 1.0)), 0.0)
    h_ref[...] = h
    dinv_ref[...] = dinv
    m0_ref[...] = (jnp.dot(h, wg0_ref[...], preferred_element_type=jnp.float32)
                   * dinv)


def _layer_mid_body(p_ref, h_ref, dinv_ref, bg_ref, gamma_ref, beta_ref,
                    wgn_ref, hout_ref, mout_ref):
    dinv = dinv_ref[...]
    agg = (p_ref[0] + p_ref[1]) * dinv + bg_ref[...]
    mu = jnp.mean(agg, axis=0, keepdims=True)
    var = jnp.mean((agg - mu) ** 2, axis=0, keepdims=True)
    hn = (agg - mu) / jnp.sqrt(var + 1e-5) * gamma_ref[...] + beta_ref[...]
    h = h_ref[...] + jnp.maximum(hn, 0.0)
    hout_ref[...] = h
    mout_ref[...] = (jnp.dot(h, wgn_ref[...],
                             preferred_element_type=jnp.float32) * dinv)


def _layer_last_body(p_ref, h_ref, dinv_ref, bg_ref, gamma_ref, beta_ref,
                     hout_ref):
    dinv = dinv_ref[...]
    agg = (p_ref[0] + p_ref[1]) * dinv + bg_ref[...]
    mu = jnp.mean(agg, axis=0, keepdims=True)
    var = jnp.mean((agg - mu) ** 2, axis=0, keepdims=True)
    hn = (agg - mu) / jnp.sqrt(var + 1e-5) * gamma_ref[...] + beta_ref[...]
    hout_ref[...] = h_ref[...] + jnp.maximum(hn, 0.0)


def _readout_body(h_ref, batch_ref, wr0_ref, br0_ref, wr1_ref, br1_ref,
                  wr2_ref, br2_ref, y_ref):
    onehot = (batch_ref[...] ==
              lax.broadcasted_iota(jnp.int32, (N, NG), 1)).astype(jnp.float32)
    g = lax.dot_general(onehot, h_ref[...], (((0,), (0,)), ((), ())),
                        preferred_element_type=jnp.float32)
    y = jnp.maximum(
        jnp.dot(g, wr0_ref[...], preferred_element_type=jnp.float32)
        + br0_ref[...], 0.0)
    y = jnp.maximum(
        jnp.dot(y, wr1_ref[...], preferred_element_type=jnp.float32)
        + br1_ref[...], 0.0)
    y_ref[...] = (jnp.dot(y, wr2_ref[...], preferred_element_type=jnp.float32)
                  + br2_ref[...])


_F = jnp.float32


def _enc_tc(x, We1, be1, We2, be2, Wg0, degp):
    return pl.pallas_call(
        _enc_body,
        out_shape=[jax.ShapeDtypeStruct((N, D), _F),
                   jax.ShapeDtypeStruct((N, D), _F),
                   jax.ShapeDtypeStruct((N, 1), _F)],
    )(x, We1, be1.reshape(1, D), We2, be2.reshape(1, D), Wg0, degp)


def _layer_mid_tc(p, h, dinv, bg, gamma, beta, Wgn):
    return pl.pallas_call(
        _layer_mid_body,
        out_shape=[jax.ShapeDtypeStruct((N, D), _F),
                   jax.ShapeDtypeStruct((N, D), _F)],
    )(p, h, dinv, bg.reshape(1, D), gamma.reshape(1, D), beta.reshape(1, D),
      Wgn)


def _layer_last_tc(p, h, dinv, bg, gamma, beta):
    return pl.pallas_call(
        _layer_last_body,
        out_shape=jax.ShapeDtypeStruct((N, D), _F),
    )(p, h, dinv, bg.reshape(1, D), gamma.reshape(1, D), beta.reshape(1, D))


def _readout_tc(h, batch, Wr0, br0, Wr1, br1, Wr2, br2):
    return pl.pallas_call(
        _readout_body,
        out_shape=jax.ShapeDtypeStruct((NG, 1), _F),
    )(h, batch.reshape(N, 1), Wr0, br0.reshape(1, D // 2),
      Wr1, br1.reshape(1, D // 4), Wr2, br2.reshape(1, 1))


def kernel(x, edge_index, batch, We1, be1, We2, be2, Wg, bg, gamma, beta,
           Wr0, br0, Wr1, br1, Wr2, br2):
    src = edge_index[0]
    dst = edge_index[1]
    zeros_nd = jnp.zeros((NS, RPT, D), _F)
    ones_chd = jnp.ones((CH, D), _F)
    _deg_sc, _agg_sc = _sc_kernels()

    degp = _deg_sc(dst, zeros_nd, ones_chd).reshape(NC, N, D)
    h, m, dinv = _enc_tc(x, We1, be1, We2, be2, Wg[0], degp)

    for l in range(NL):
        p = _agg_sc(m, src, dst, zeros_nd).reshape(NC, N, D)
        if l < NL - 1:
            h, m = _layer_mid_tc(p, h, dinv, bg[l], gamma[l], beta[l],
                                 Wg[l + 1])
        else:
            h = _layer_last_tc(p, h, dinv, bg[l], gamma[l], beta[l])

    return _readout_tc(h, batch, Wr0, br0, Wr1, br1, Wr2, br2)


# speedup vs baseline: 14.0596x; 1.6957x over previous
import functools

import jax
import jax.numpy as jnp
from jax import lax
from jax.experimental import pallas as pl
from jax.experimental.pallas import tpu as pltpu
from jax.experimental.pallas import tpu_sc as plsc

N = 10000
D = 128
E = 320000
NG = 64
NL = 3
NC = 2
NS = 16
NW = NC * NS
EPW = E // NW
CH = 80
NCH = EPW // CH
RPT = N // NS


@functools.cache
def _sc_kernels():
  mesh = plsc.VectorSubcoreMesh(core_axis_name="c", subcore_axis_name="s",
                                num_cores=NC, num_subcores=NS)

  @functools.partial(
      pl.kernel,
      out_type=jax.ShapeDtypeStruct((NW, RPT, D), jnp.float32),
      mesh=mesh,
      scratch_types=[
          pltpu.VMEM((CH,), jnp.int32),
          pltpu.VMEM((CH, D), jnp.float32),
          pltpu.VMEM_SHARED((N, D), jnp.float32),
      ],
  )
  def _deg_sc(dst_hbm, zeros_hbm, ones_hbm, out_hbm, didx, ones_v, deg_sh):
      c = lax.axis_index("c")
      s = lax.axis_index("s")
      wid = c * NS + s
      pltpu.sync_copy(zeros_hbm.at[s], deg_sh.at[pl.ds(s * RPT, RPT)])
      pltpu.sync_copy(ones_hbm, ones_v)
      plsc.subcore_barrier()
      base = wid * EPW

      def body(i, carry):
        pltpu.sync_copy(dst_hbm.at[pl.ds(base + i * CH, CH)], didx)
        pltpu.sync_copy(ones_v, deg_sh.at[didx], add=True)
        return carry

      lax.fori_loop(0, NCH, body, 0)
      plsc.subcore_barrier()
      pltpu.sync_copy(deg_sh.at[pl.ds(s * RPT, RPT)], out_hbm.at[wid])

  @functools.partial(
      pl.kernel,
      out_type=jax.ShapeDtypeStruct((NW, RPT, D), jnp.float32),
      mesh=mesh,
      scratch_types=[
          pltpu.VMEM((CH,), jnp.int32),
          pltpu.VMEM((CH,), jnp.int32),
          pltpu.VMEM((CH,), jnp.int32),
          pltpu.VMEM((CH,), jnp.int32),
          pltpu.VMEM((CH, D), jnp.float32),
          pltpu.VMEM((CH, D), jnp.float32),
          pltpu.VMEM_SHARED((N, D), jnp.float32),
          pltpu.SemaphoreType.DMA,
          pltpu.SemaphoreType.DMA,
          pltpu.SemaphoreType.DMA,
          pltpu.SemaphoreType.DMA,
          pltpu.SemaphoreType.DMA,
          pltpu.SemaphoreType.DMA,
      ],
  )
  def _agg_sc(m_hbm, src_hbm, dst_hbm, zeros_hbm, out_hbm,
              sidx0, didx0, sidx1, didx1, rows0, rows1, agg_sh,
              ss0, ds0, gs0, ss1, ds1, gs1):
      c = lax.axis_index("c")
      s = lax.axis_index("s")
      wid = c * NS + s
      pltpu.sync_copy(zeros_hbm.at[s], agg_sh.at[pl.ds(s * RPT, RPT)])
      plsc.subcore_barrier()
      base = wid * EPW

      def chunk(i):
          return src_hbm.at[pl.ds(base + i * CH, CH)], dst_hbm.at[pl.ds(base + i * CH, CH)]

      def start_idx(i, sidx, didx, ssem, dsem):
          sl_s, sl_d = chunk(i)
          pltpu.async_copy(sl_s, sidx, ssem)
          pltpu.async_copy(sl_d, didx, dsem)

      def wait_idx(i, sidx, didx, ssem, dsem):
          sl_s, sl_d = chunk(i)
          pltpu.make_async_copy(sl_s, sidx, ssem).wait()
          pltpu.make_async_copy(sl_d, didx, dsem).wait()

      start_idx(0, sidx0, didx0, ss0, ds0)
      start_idx(1, sidx1, didx1, ss1, ds1)
      pltpu.make_async_copy(src_hbm.at[pl.ds(base, CH)], sidx0, ss0).wait()
      pltpu.async_copy(m_hbm.at[sidx0], rows0, gs0)

      def body(t, carry):
          i0 = 2 * t
          i1 = 2 * t + 1
          pltpu.make_async_copy(src_hbm.at[pl.ds(base + i1 * CH, CH)],
                                sidx1, ss1).wait()
          pltpu.async_copy(m_hbm.at[sidx1], rows1, gs1)
          pltpu.make_async_copy(m_hbm.at[sidx0], rows0, gs0).wait()
          pltpu.make_async_copy(dst_hbm.at[pl.ds(base + i0 * CH, CH)],
                                didx0, ds0).wait()
          pltpu.sync_copy(rows0, agg_sh.at[didx0], add=True)
          start_idx(i0 + 2, sidx0, didx0, ss0, ds0)
          pltpu.make_async_copy(m_hbm.at[sidx1], rows1, gs1).wait()
          pltpu.make_async_copy(dst_hbm.at[pl.ds(base + i1 * CH, CH)],
                                didx1, ds1).wait()
          pltpu.sync_copy(rows1, agg_sh.at[didx1], add=True)

          @pl.when(i1 + 2 < NCH)
          def _():
              start_idx(i1 + 2, sidx1, didx1, ss1, ds1)

          pltpu.make_async_copy(src_hbm.at[pl.ds(base + (i0 + 2) * CH, CH)],
                                sidx0, ss0).wait()
          pltpu.async_copy(m_hbm.at[sidx0], rows0, gs0)
          return carry

      lax.fori_loop(0, (NCH - 1) // 2, body, 0)
      last = NCH - 1
      pltpu.make_async_copy(m_hbm.at[sidx0], rows0, gs0).wait()
      pltpu.make_async_copy(dst_hbm.at[pl.ds(base + last * CH, CH)],
                            didx0, ds0).wait()
      pltpu.sync_copy(rows0, agg_sh.at[didx0], add=True)
      plsc.subcore_barrier()
      pltpu.sync_copy(agg_sh.at[pl.ds(s * RPT, RPT)], out_hbm.at[wid])

  return _deg_sc, _agg_sc


def _enc_body(x_ref, we1_ref, be1_ref, we2_ref, be2_ref, wg0_ref, degp_ref,
              h_ref, m0_ref, dinv_ref):
    x = x_ref[...]
    h1 = jnp.maximum(
        jnp.dot(x, we1_ref[...], preferred_element_type=jnp.float32)
        + be1_ref[...], 0.0)
    h = (jnp.dot(h1, we2_ref[...], preferred_element_type=jnp.float32)
         + be2_ref[...])
    degw = degp_ref[0] + degp_ref[1]
    deg = jnp.sum(degw, axis=1, keepdims=True) * (1.0 / D)
    dinv = jnp.where(deg > 0.0, lax.rsqrt(jnp.maximum(deg, 1.0)), 0.0)
    h_ref[...] = h
    dinv_ref[...] = dinv
    m0_ref[...] = (jnp.dot(h, wg0_ref[...], preferred_element_type=jnp.float32)
                   * dinv)


def _layer_mid_body(p_ref, h_ref, dinv_ref, bg_ref, gamma_ref, beta_ref,
                    wgn_ref, hout_ref, mout_ref):
    dinv = dinv_ref[...]
    agg = (p_ref[0] + p_ref[1]) * dinv + bg_ref[...]
    mu = jnp.mean(agg, axis=0, keepdims=True)
    var = jnp.mean((agg - mu) ** 2, axis=0, keepdims=True)
    hn = (agg - mu) / jnp.sqrt(var + 1e-5) * gamma_ref[...] + beta_ref[...]
    h = h_ref[...] + jnp.maximum(hn, 0.0)
    hout_ref[...] = h
    mout_ref[...] = (jnp.dot(h, wgn_ref[...],
                             preferred_element_type=jnp.float32) * dinv)


def _layer_last_body(p_ref, h_ref, dinv_ref, bg_ref, gamma_ref, beta_ref,
                     hout_ref):
    dinv = dinv_ref[...]
    agg = (p_ref[0] + p_ref[1]) * dinv + bg_ref[...]
    mu = jnp.mean(agg, axis=0, keepdims=True)
    var = jnp.mean((agg - mu) ** 2, axis=0, keepdims=True)
    hn = (agg - mu) / jnp.sqrt(var + 1e-5) * gamma_ref[...] + beta_ref[...]
    hout_ref[...] = h_ref[...] + jnp.maximum(hn, 0.0)


def _readout_body(h_ref, batch_ref, wr0_ref, br0_ref, wr1_ref, br1_ref,
                  wr2_ref, br2_ref, y_ref):
    onehot = (batch_ref[...] ==
              lax.broadcasted_iota(jnp.int32, (N, NG), 1)).astype(jnp.float32)
    g = lax.dot_general(onehot, h_ref[...], (((0,), (0,)), ((), ())),
                        preferred_element_type=jnp.float32,
                        precision=lax.Precision.HIGHEST)
    y = jnp.maximum(
        jnp.dot(g, wr0_ref[...], preferred_element_type=jnp.float32)
        + br0_ref[...], 0.0)
    y = jnp.maximum(
        jnp.dot(y, wr1_ref[...], preferred_element_type=jnp.float32)
        + br1_ref[...], 0.0)
    y_ref[...] = (jnp.dot(y, wr2_ref[...], preferred_element_type=jnp.float32)
                  + br2_ref[...])


_F = jnp.float32


def _enc_tc(x, We1, be1, We2, be2, Wg0, degp):
    return pl.pallas_call(
        _enc_body,
        out_shape=[jax.ShapeDtypeStruct((N, D), _F),
                   jax.ShapeDtypeStruct((N, D), _F),
                   jax.ShapeDtypeStruct((N, 1), _F)],
    )(x, We1, be1.reshape(1, D), We2, be2.reshape(1, D), Wg0, degp)


def _layer_mid_tc(p, h, dinv, bg, gamma, beta, Wgn):
    return pl.pallas_call(
        _layer_mid_body,
        out_shape=[jax.ShapeDtypeStruct((N, D), _F),
                   jax.ShapeDtypeStruct((N, D), _F)],
    )(p, h, dinv, bg.reshape(1, D), gamma.reshape(1, D), beta.reshape(1, D),
      Wgn)


def _layer_last_tc(p, h, dinv, bg, gamma, beta):
    return pl.pallas_call(
        _layer_last_body,
        out_shape=jax.ShapeDtypeStruct((N, D), _F),
    )(p, h, dinv, bg.reshape(1, D), gamma.reshape(1, D), beta.reshape(1, D))


def _readout_tc(h, batch, Wr0, br0, Wr1, br1, Wr2, br2):
    return pl.pallas_call(
        _readout_body,
        out_shape=jax.ShapeDtypeStruct((NG, 1), _F),
    )(h, batch.reshape(N, 1), Wr0, br0.reshape(1, D // 2),
      Wr1, br1.reshape(1, D // 4), Wr2, br2.reshape(1, 1))


def kernel(x, edge_index, batch, We1, be1, We2, be2, Wg, bg, gamma, beta,
           Wr0, br0, Wr1, br1, Wr2, br2):
    src = edge_index[0]
    dst = edge_index[1]
    zeros_nd = jnp.zeros((NS, RPT, D), _F)
    ones_chd = jnp.ones((CH, D), _F)
    _deg_sc, _agg_sc = _sc_kernels()

    degp = _deg_sc(dst, zeros_nd, ones_chd).reshape(NC, N, D)
    h, m, dinv = _enc_tc(x, We1, be1, We2, be2, Wg[0], degp)

    for l in range(NL):
        p = _agg_sc(m, src, dst, zeros_nd).reshape(NC, N, D)
        if l < NL - 1:
            h, m = _layer_mid_tc(p, h, dinv, bg[l], gamma[l], beta[l],
                                 Wg[l + 1])
        else:
            h = _layer_last_tc(p, h, dinv, bg[l], gamma[l], beta[l])

    return _readout_tc(h, batch, Wr0, br0, Wr1, br1, Wr2, br2)


# speedup vs baseline: 15.2963x; 1.0880x over previous
import functools

import jax
import jax.numpy as jnp
from jax import lax
from jax.experimental import pallas as pl
from jax.experimental.pallas import tpu as pltpu
from jax.experimental.pallas import tpu_sc as plsc

N = 10000
D = 128
E = 320000
NG = 64
NL = 3
NC = 2
NS = 16
NW = NC * NS
EPW = E // NW
CH = 80
NCH = EPW // CH
RPT = N // NS


@functools.cache
def _sc_kernels():
  mesh = plsc.VectorSubcoreMesh(core_axis_name="c", subcore_axis_name="s",
                                num_cores=NC, num_subcores=NS)

  @functools.partial(
      pl.kernel,
      out_type=jax.ShapeDtypeStruct((NW, RPT, D), jnp.float32),
      mesh=mesh,
      scratch_types=[
          pltpu.VMEM((CH,), jnp.int32),
          pltpu.VMEM((CH,), jnp.int32),
          pltpu.VMEM((CH, D), jnp.float32),
          pltpu.VMEM_SHARED((N, D), jnp.float32),
          pltpu.SemaphoreType.DMA,
          pltpu.SemaphoreType.DMA,
      ],
  )
  def _deg_sc(dst_hbm, zeros_hbm, ones_hbm, out_hbm, didx0, didx1, ones_v,
              deg_sh, ds0, ds1):
      c = lax.axis_index("c")
      s = lax.axis_index("s")
      wid = c * NS + s
      pltpu.sync_copy(zeros_hbm.at[s], deg_sh.at[pl.ds(s * RPT, RPT)])
      pltpu.sync_copy(ones_hbm, ones_v)
      plsc.subcore_barrier()
      base = wid * EPW

      def dchunk(i):
          return dst_hbm.at[pl.ds(base + i * CH, CH)]

      pltpu.async_copy(dchunk(0), didx0, ds0)
      pltpu.async_copy(dchunk(1), didx1, ds1)

      def body(t, carry):
          i0 = 2 * t
          i1 = 2 * t + 1
          pltpu.make_async_copy(dchunk(i0), didx0, ds0).wait()
          pltpu.sync_copy(ones_v, deg_sh.at[didx0], add=True)
          pltpu.async_copy(dchunk(i0 + 2), didx0, ds0)
          pltpu.make_async_copy(dchunk(i1), didx1, ds1).wait()
          pltpu.sync_copy(ones_v, deg_sh.at[didx1], add=True)

          @pl.when(i1 + 2 < NCH)
          def _():
              pltpu.async_copy(dchunk(i1 + 2), didx1, ds1)

          return carry

      lax.fori_loop(0, (NCH - 1) // 2, body, 0)
      last = NCH - 1
      pltpu.make_async_copy(dchunk(last), didx0, ds0).wait()
      pltpu.sync_copy(ones_v, deg_sh.at[didx0], add=True)
      plsc.subcore_barrier()
      pltpu.sync_copy(deg_sh.at[pl.ds(s * RPT, RPT)], out_hbm.at[wid])


  @functools.partial(
      pl.kernel,
      out_type=jax.ShapeDtypeStruct((NW, RPT, D), jnp.float32),
      mesh=mesh,
      scratch_types=[
          pltpu.VMEM((CH,), jnp.int32),
          pltpu.VMEM((CH,), jnp.int32),
          pltpu.VMEM((CH,), jnp.int32),
          pltpu.VMEM((CH,), jnp.int32),
          pltpu.VMEM((CH, D), jnp.float32),
          pltpu.VMEM((CH, D), jnp.float32),
          pltpu.VMEM_SHARED((N, D), jnp.float32),
          pltpu.SemaphoreType.DMA,
          pltpu.SemaphoreType.DMA,
          pltpu.SemaphoreType.DMA,
          pltpu.SemaphoreType.DMA,
          pltpu.SemaphoreType.DMA,
          pltpu.SemaphoreType.DMA,
      ],
  )
  def _agg_sc(m_hbm, src_hbm, dst_hbm, zeros_hbm, out_hbm,
              sidx0, didx0, sidx1, didx1, rows0, rows1, agg_sh,
              ss0, ds0, gs0, ss1, ds1, gs1):
      c = lax.axis_index("c")
      s = lax.axis_index("s")
      wid = c * NS + s
      pltpu.sync_copy(zeros_hbm.at[s], agg_sh.at[pl.ds(s * RPT, RPT)])
      plsc.subcore_barrier()
      base = wid * EPW

      def chunk(i):
          return src_hbm.at[pl.ds(base + i * CH, CH)], dst_hbm.at[pl.ds(base + i * CH, CH)]

      def start_idx(i, sidx, didx, ssem, dsem):
          sl_s, sl_d = chunk(i)
          pltpu.async_copy(sl_s, sidx, ssem)
          pltpu.async_copy(sl_d, didx, dsem)

      def wait_idx(i, sidx, didx, ssem, dsem):
          sl_s, sl_d = chunk(i)
          pltpu.make_async_copy(sl_s, sidx, ssem).wait()
          pltpu.make_async_copy(sl_d, didx, dsem).wait()

      start_idx(0, sidx0, didx0, ss0, ds0)
      start_idx(1, sidx1, didx1, ss1, ds1)
      pltpu.make_async_copy(src_hbm.at[pl.ds(base, CH)], sidx0, ss0).wait()
      pltpu.async_copy(m_hbm.at[sidx0], rows0, gs0)

      def body(t, carry):
          i0 = 2 * t
          i1 = 2 * t + 1
          pltpu.make_async_copy(src_hbm.at[pl.ds(base + i1 * CH, CH)],
                                sidx1, ss1).wait()
          pltpu.async_copy(m_hbm.at[sidx1], rows1, gs1)
          pltpu.make_async_copy(m_hbm.at[sidx0], rows0, gs0).wait()
          pltpu.make_async_copy(dst_hbm.at[pl.ds(base + i0 * CH, CH)],
                                didx0, ds0).wait()
          pltpu.sync_copy(rows0, agg_sh.at[didx0], add=True)
          start_idx(i0 + 2, sidx0, didx0, ss0, ds0)
          pltpu.make_async_copy(m_hbm.at[sidx1], rows1, gs1).wait()
          pltpu.make_async_copy(dst_hbm.at[pl.ds(base + i1 * CH, CH)],
                                didx1, ds1).wait()
          pltpu.sync_copy(rows1, agg_sh.at[didx1], add=True)

          @pl.when(i1 + 2 < NCH)
          def _():
              start_idx(i1 + 2, sidx1, didx1, ss1, ds1)

          pltpu.make_async_copy(src_hbm.at[pl.ds(base + (i0 + 2) * CH, CH)],
                                sidx0, ss0).wait()
          pltpu.async_copy(m_hbm.at[sidx0], rows0, gs0)
          return carry

      lax.fori_loop(0, (NCH - 1) // 2, body, 0)
      last = NCH - 1
      pltpu.make_async_copy(m_hbm.at[sidx0], rows0, gs0).wait()
      pltpu.make_async_copy(dst_hbm.at[pl.ds(base + last * CH, CH)],
                            didx0, ds0).wait()
      pltpu.sync_copy(rows0, agg_sh.at[didx0], add=True)
      plsc.subcore_barrier()
      pltpu.sync_copy(agg_sh.at[pl.ds(s * RPT, RPT)], out_hbm.at[wid])

  return _deg_sc, _agg_sc


def _enc_body(x_ref, we1_ref, be1_ref, we2_ref, be2_ref, wg0_ref, degp_ref,
              h_ref, m0_ref, dinv_ref):
    x = x_ref[...]
    h1 = jnp.maximum(
        jnp.dot(x, we1_ref[...], preferred_element_type=jnp.float32)
        + be1_ref[...], 0.0)
    h = (jnp.dot(h1, we2_ref[...], preferred_element_type=jnp.float32)
         + be2_ref[...])
    degw = degp_ref[0] + degp_ref[1]
    deg = jnp.sum(degw, axis=1, keepdims=True) * (1.0 / D)
    dinv = jnp.where(deg > 0.0, lax.rsqrt(jnp.maximum(deg, 1.0)), 0.0)
    h_ref[...] = h
    dinv_ref[...] = dinv
    m0_ref[...] = (jnp.dot(h, wg0_ref[...], preferred_element_type=jnp.float32)
                   * dinv)


def _layer_mid_body(p_ref, h_ref, dinv_ref, bg_ref, gamma_ref, beta_ref,
                    wgn_ref, hout_ref, mout_ref):
    dinv = dinv_ref[...]
    agg = (p_ref[0] + p_ref[1]) * dinv + bg_ref[...]
    mu = jnp.mean(agg, axis=0, keepdims=True)
    var = jnp.mean((agg - mu) ** 2, axis=0, keepdims=True)
    hn = (agg - mu) / jnp.sqrt(var + 1e-5) * gamma_ref[...] + beta_ref[...]
    h = h_ref[...] + jnp.maximum(hn, 0.0)
    hout_ref[...] = h
    mout_ref[...] = (jnp.dot(h, wgn_ref[...],
                             preferred_element_type=jnp.float32) * dinv)


def _last_readout_body(p_ref, h_ref, dinv_ref, bg_ref, gamma_ref, beta_ref,
                       batch_ref, wr0_ref, br0_ref, wr1_ref, br1_ref,
                       wr2_ref, br2_ref, y_ref):
    dinv = dinv_ref[...]
    agg = (p_ref[0] + p_ref[1]) * dinv + bg_ref[...]
    mu = jnp.mean(agg, axis=0, keepdims=True)
    var = jnp.mean((agg - mu) ** 2, axis=0, keepdims=True)
    hn = (agg - mu) / jnp.sqrt(var + 1e-5) * gamma_ref[...] + beta_ref[...]
    h = h_ref[...] + jnp.maximum(hn, 0.0)
    onehot = (batch_ref[...] ==
              lax.broadcasted_iota(jnp.int32, (N, NG), 1)).astype(jnp.float32)
    g = lax.dot_general(onehot, h, (((0,), (0,)), ((), ())),
                        preferred_element_type=jnp.float32,
                        precision=lax.Precision.HIGHEST)
    y = jnp.maximum(
        jnp.dot(g, wr0_ref[...], preferred_element_type=jnp.float32)
        + br0_ref[...], 0.0)
    y = jnp.maximum(
        jnp.dot(y, wr1_ref[...], preferred_element_type=jnp.float32)
        + br1_ref[...], 0.0)
    y_ref[...] = (jnp.dot(y, wr2_ref[...], preferred_element_type=jnp.float32)
                  + br2_ref[...])


_F = jnp.float32


def _enc_tc(x, We1, be1, We2, be2, Wg0, degp):
    return pl.pallas_call(
        _enc_body,
        out_shape=[jax.ShapeDtypeStruct((N, D), _F),
                   jax.ShapeDtypeStruct((N, D), _F),
                   jax.ShapeDtypeStruct((N, 1), _F)],
    )(x, We1, be1.reshape(1, D), We2, be2.reshape(1, D), Wg0, degp)


def _layer_mid_tc(p, h, dinv, bg, gamma, beta, Wgn):
    return pl.pallas_call(
        _layer_mid_body,
        out_shape=[jax.ShapeDtypeStruct((N, D), _F),
                   jax.ShapeDtypeStruct((N, D), _F)],
    )(p, h, dinv, bg.reshape(1, D), gamma.reshape(1, D), beta.reshape(1, D),
      Wgn)


def _last_readout_tc(p, h, dinv, bg, gamma, beta, batch,
                     Wr0, br0, Wr1, br1, Wr2, br2):
    return pl.pallas_call(
        _last_readout_body,
        out_shape=jax.ShapeDtypeStruct((NG, 1), _F),
    )(p, h, dinv, bg.reshape(1, D), gamma.reshape(1, D), beta.reshape(1, D),
      batch.reshape(N, 1), Wr0, br0.reshape(1, D // 2),
      Wr1, br1.reshape(1, D // 4), Wr2, br2.reshape(1, 1))


def kernel(x, edge_index, batch, We1, be1, We2, be2, Wg, bg, gamma, beta,
           Wr0, br0, Wr1, br1, Wr2, br2):
    src = edge_index[0]
    dst = edge_index[1]
    zeros_nd = jnp.zeros((NS, RPT, D), _F)
    ones_chd = jnp.ones((CH, D), _F)
    _deg_sc, _agg_sc = _sc_kernels()

    degp = _deg_sc(dst, zeros_nd, ones_chd).reshape(NC, N, D)
    h, m, dinv = _enc_tc(x, We1, be1, We2, be2, Wg[0], degp)

    for l in range(NL):
        p = _agg_sc(m, src, dst, zeros_nd).reshape(NC, N, D)
        if l < NL - 1:
            h, m = _layer_mid_tc(p, h, dinv, bg[l], gamma[l], beta[l],
                                 Wg[l + 1])
        else:
            return _last_readout_tc(p, h, dinv, bg[l], gamma[l], beta[l],
                                    batch, Wr0, br0, Wr1, br1, Wr2, br2)
